# Initial kernel scaffold; baseline (speedup 1.0000x reference)
import jax
import jax.numpy as jnp
from jax.experimental import pallas as pl


def kernel(q_src, k_src, pe, q_w, k_w, v_w, o_w, sel_v_w, sel_o_w):
    raise NotImplementedError("write your pallas kernel here")


# speedup vs baseline: 5.7618x; 5.7618x over previous
import functools

import jax
import jax.numpy as jnp
from jax.experimental import pallas as pl
from jax.experimental.pallas import tpu as pltpu

_BF = jnp.bfloat16


def _dot(a, b):
    return jax.lax.dot_general(
        a.astype(_BF), b.astype(_BF), (((1,), (0,)), ((), ())),
        preferred_element_type=jnp.float32)


def _dot_t(a, b):
    return jax.lax.dot_general(
        a.astype(_BF), b.astype(_BF), (((1,), (1,)), ((), ())),
        preferred_element_type=jnp.float32)


def _top2_gates(raw, H, E):
    val = 1.0 / (1.0 + jnp.exp(-raw))
    bs = raw.shape[0]
    it = jax.lax.broadcasted_iota(jnp.int32, (bs, E), 1)
    outs = []
    for h in range(H):
        v8 = val[:, h * E:(h + 1) * E]
        m1 = jnp.max(v8, axis=1, keepdims=True)
        i1 = jnp.min(jnp.where(v8 == m1, it, E + 1), axis=1, keepdims=True)
        msk1 = it == i1
        v2 = jnp.where(msk1, -1.0, v8)
        m2 = jnp.max(v2, axis=1, keepdims=True)
        i2 = jnp.min(jnp.where(v2 == m2, it, E + 1), axis=1, keepdims=True)
        msk2 = it == i2
        outs.append(jnp.where(msk1, m1, 0.0) + jnp.where(msk2, m2, 0.0))
    return jnp.concatenate(outs, axis=1)


def _proj_body(H, E, P, qs_ref, ks_ref, pe_ref, wq_ref, wk_ref, wv_ref,
               sv_ref, so_ref, qo_ref, ko_ref, kpo_ref, vo_ref, go_ref):
    x_q = qs_ref[...]
    x_k = ks_ref[...]
    q_big = _dot(x_q, wq_ref[...])
    k_big = _dot(x_k, wk_ref[...])
    kp_big = _dot(pe_ref[...], wk_ref[...])
    gv = _top2_gates(_dot(x_k, sv_ref[...]), H, E)
    go_ref[...] = _top2_gates(_dot(x_q, so_ref[...]), H, E)
    v_all = _dot(x_k, wv_ref[...])
    for h in range(H):
        qo_ref[h] = q_big[:, h * P:(h + 1) * P]
        ko_ref[h] = k_big[:, h * P:(h + 1) * P]
        kpo_ref[h] = kp_big[:, h * P:(h + 1) * P]
        acc = None
        for e in range(E):
            j = h * E + e
            piece = v_all[:, j * P:(j + 1) * P] * gv[:, j:j + 1]
            acc = piece if acc is None else acc + piece
        vo_ref[h] = acc


def _attn_body(S, BQ, P, scale, q_ref, k_ref, v_ref, kp_ref, o_ref):
    qb = pl.program_id(1)
    i0 = qb * BQ
    q = q_ref[0]
    content = _dot_t(q, k_ref[0])
    pmat = _dot_t(q, kp_ref[0])
    pos = pltpu.roll(pmat, i0 + 1, 1)
    pos = pltpu.roll(pos, 0, 1, stride=1, stride_axis=0)
    it = jax.lax.broadcasted_iota(jnp.int32, (BQ, S), 1)
    ir = jax.lax.broadcasted_iota(jnp.int32, (BQ, S), 0) + i0
    x = jnp.where(it <= ir, (content + pos) * scale, -1e30)
    m = jnp.max(x, axis=1, keepdims=True)
    p = jnp.exp(x - m)
    att = p / jnp.sum(p, axis=1, keepdims=True)
    o_ref[0] = _dot(att, v_ref[0])


def _out_body(H, E, P, ctx_ref, go_ref, wo_ref, out_ref):
    go = go_ref[...]
    acc = None
    for e in range(E):
        parts = [ctx_ref[h] * go[:, h * E + e:h * E + e + 1]
                 for h in range(H)]
        ge = jnp.concatenate(parts, axis=1)
        term = _dot(ge, wo_ref[e])
        acc = term if acc is None else acc + term
    out_ref[...] = acc


def kernel(q_src, k_src, pe, q_w, k_w, v_w, o_w, sel_v_w, sel_o_w):
    B, S, D = q_src.shape
    H, _, P = q_w.shape
    E = v_w.shape[1]
    HP = H * P
    HE = H * E

    qs = q_src[0]
    ks = k_src[0]
    pes = pe[:S]
    wq = q_w.transpose(1, 0, 2).reshape(D, HP)
    wk = k_w.transpose(1, 0, 2).reshape(D, HP)
    wv = v_w.transpose(2, 0, 1, 3).reshape(D, H * E * P)
    wo = o_w.transpose(1, 0, 2, 3).reshape(E, HP, D)

    BS = 128
    nS = S // BS
    f32 = jnp.float32
    q_p, k_p, kpe_p, v_p, go_p = pl.pallas_call(
        functools.partial(_proj_body, H, E, P),
        grid=(nS,),
        in_specs=[
            pl.BlockSpec((BS, D), lambda i: (i, 0)),
            pl.BlockSpec((BS, D), lambda i: (i, 0)),
            pl.BlockSpec((BS, D), lambda i: (i, 0)),
            pl.BlockSpec((D, HP), lambda i: (0, 0)),
            pl.BlockSpec((D, HP), lambda i: (0, 0)),
            pl.BlockSpec((D, H * E * P), lambda i: (0, 0)),
            pl.BlockSpec((D, HE), lambda i: (0, 0)),
            pl.BlockSpec((D, HE), lambda i: (0, 0)),
        ],
        out_specs=[
            pl.BlockSpec((H, BS, P), lambda i: (0, i, 0)),
            pl.BlockSpec((H, BS, P), lambda i: (0, i, 0)),
            pl.BlockSpec((H, BS, P), lambda i: (0, i, 0)),
            pl.BlockSpec((H, BS, P), lambda i: (0, i, 0)),
            pl.BlockSpec((BS, HE), lambda i: (i, 0)),
        ],
        out_shape=[
            jax.ShapeDtypeStruct((H, S, P), f32),
            jax.ShapeDtypeStruct((H, S, P), f32),
            jax.ShapeDtypeStruct((H, S, P), f32),
            jax.ShapeDtypeStruct((H, S, P), f32),
            jax.ShapeDtypeStruct((S, HE), f32),
        ],
    )(qs, ks, pes, wq, wk, wv, sel_v_w, sel_o_w)

    BQ = 512
    nQ = S // BQ
    scale = 1.0 / float(P) ** 0.5
    ctx = pl.pallas_call(
        functools.partial(_attn_body, S, BQ, P, scale),
        grid=(H, nQ),
        in_specs=[
            pl.BlockSpec((1, BQ, P), lambda h, i: (h, i, 0)),
            pl.BlockSpec((1, S, P), lambda h, i: (h, 0, 0)),
            pl.BlockSpec((1, S, P), lambda h, i: (h, 0, 0)),
            pl.BlockSpec((1, S, P), lambda h, i: (h, 0, 0)),
        ],
        out_specs=pl.BlockSpec((1, BQ, P), lambda h, i: (h, i, 0)),
        out_shape=jax.ShapeDtypeStruct((H, S, P), f32),
    )(q_p, k_p, v_p, kpe_p)

    BO = 512
    nO = S // BO
    out = pl.pallas_call(
        functools.partial(_out_body, H, E, P),
        grid=(nO,),
        in_specs=[
            pl.BlockSpec((H, BO, P), lambda i: (0, i, 0)),
            pl.BlockSpec((BO, HE), lambda i: (i, 0)),
            pl.BlockSpec((E, HP, D), lambda i: (0, 0, 0)),
        ],
        out_specs=pl.BlockSpec((BO, D), lambda i: (i, 0)),
        out_shape=jax.ShapeDtypeStruct((S, D), f32),
    )(ctx, go_p, wo)

    return out.reshape(B, S, D)


# speedup vs baseline: 8.7933x; 1.5261x over previous
import functools

import jax
import jax.numpy as jnp
from jax.experimental import pallas as pl
from jax.experimental.pallas import tpu as pltpu

_BF = jnp.bfloat16
_GL = 128


def _dot(a, b):
    return jax.lax.dot_general(
        a.astype(_BF), b.astype(_BF), (((1,), (0,)), ((), ())),
        preferred_element_type=jnp.float32)


def _dot_t(a, b):
    return jax.lax.dot_general(
        a.astype(_BF), b.astype(_BF), (((1,), (1,)), ((), ())),
        preferred_element_type=jnp.float32)


def _spread(e, H, E, P, D):
    jr = jax.lax.broadcasted_iota(jnp.int32, (_GL, H * P), 0)
    jc = jax.lax.broadcasted_iota(jnp.int32, (_GL, H * P), 1)
    return jnp.where(jr == (jc // P) * E + e, 1.0, 0.0).astype(jnp.float32)


def _grp_sel(m, i, ms, is_, ok):
    better = ok & ((ms > m) | ((ms == m) & (is_ < i)))
    return jnp.where(better, ms, m), jnp.where(better, is_, i)


def _grp_argmax(val, p8, it8):
    W = val.shape[1]
    pm, pi = val, it8
    sm, si = val, it8
    for k in (1, 2, 4):
        pm_s = pltpu.roll(pm, k, 1)
        pi_s = pltpu.roll(pi, k, 1)
        pm, pi = _grp_sel(pm, pi, pm_s, pi_s, p8 >= k)
        sm_s = pltpu.roll(sm, W - k, 1)
        si_s = pltpu.roll(si, W - k, 1)
        sm, si = _grp_sel(sm, si, sm_s, si_s, p8 <= 7 - k)
    return _grp_sel(pm, pi, sm, si, pm == pm)


def _top2_gates(raw):
    val = 1.0 / (1.0 + jnp.exp(-raw))
    bs = raw.shape[0]
    lane = jax.lax.broadcasted_iota(jnp.int32, (bs, _GL), 1)
    p8 = jax.lax.rem(lane, 8)
    it8 = p8.astype(jnp.float32)
    p8f = it8
    m1, i1 = _grp_argmax(val, p8f, it8)
    msk1 = it8 == i1
    val2 = jnp.where(msk1, -1.0, val)
    m2, i2 = _grp_argmax(val2, p8f, it8)
    msk2 = it8 == i2
    return jnp.where(msk1, m1, 0.0) + jnp.where(msk2, m2, 0.0)


def _proj_body(H, E, P, D, qs_ref, ks_ref, pe_ref, wqs_ref, wks_ref, wv_ref,
               qo_ref, ko_ref, kpo_ref, vo_ref, go_ref):
    HP = H * P
    x_q = qs_ref[...]
    x_k = ks_ref[...]
    qs_big = _dot(x_q, wqs_ref[...])
    ks_big = _dot(x_k, wks_ref[...])
    kp_big = _dot(pe_ref[...], wks_ref[..., :HP])
    q_big = qs_big[:, :HP]
    k_big = ks_big[:, :HP]
    gv = _top2_gates(ks_big[:, HP:])
    go_ref[...] = _top2_gates(qs_big[:, HP:])
    v_all = _dot(x_k, wv_ref[...])
    v_acc = None
    for e in range(E):
        ge = _dot(gv, _spread(e, H, E, P, D))
        piece = v_all[:, e * HP:(e + 1) * HP] * ge
        v_acc = piece if v_acc is None else v_acc + piece
    for h in range(H):
        qo_ref[h] = q_big[:, h * P:(h + 1) * P]
        ko_ref[h] = k_big[:, h * P:(h + 1) * P]
        kpo_ref[h] = kp_big[:, h * P:(h + 1) * P]
        vo_ref[h] = v_acc[:, h * P:(h + 1) * P]


def _attn_body(S, BQ, P, scale, q_ref, k_ref, v_ref, kp_ref, o_ref):
    qb = pl.program_id(1)
    i0 = qb * BQ
    q = q_ref[0]
    content = _dot_t(q, k_ref[0])
    pmat = _dot_t(q, kp_ref[0])
    pos = pltpu.roll(pmat, i0 + 1, 1)
    pos = pltpu.roll(pos, 0, 1, stride=1, stride_axis=0)
    it = jax.lax.broadcasted_iota(jnp.int32, (BQ, S), 1)
    ir = jax.lax.broadcasted_iota(jnp.int32, (BQ, S), 0) + i0
    x = jnp.where(it <= ir, (content + pos) * scale, -1e30)
    m = jnp.max(x, axis=1, keepdims=True)
    p = jnp.exp(x - m)
    att = p / jnp.sum(p, axis=1, keepdims=True)
    o_ref[0] = _dot(att, v_ref[0])


def _out_body(H, E, P, D, ctx_ref, go_ref, wo_ref, out_ref):
    ctx = ctx_ref[...]
    go = go_ref[...]
    acc = None
    for e in range(E):
        ge = _dot(go, _spread(e, H, E, P, D))
        term = _dot(ctx * ge, wo_ref[e])
        acc = term if acc is None else acc + term
    out_ref[...] = acc


def kernel(q_src, k_src, pe, q_w, k_w, v_w, o_w, sel_v_w, sel_o_w):
    B, S, D = q_src.shape
    H, _, P = q_w.shape
    E = v_w.shape[1]
    HP = H * P
    HE = H * E

    qs = q_src[0].astype(_BF)
    ks = k_src[0].astype(_BF)
    pes = pe[:S].astype(_BF)
    wq = q_w.transpose(1, 0, 2).reshape(D, HP)
    wk = k_w.transpose(1, 0, 2).reshape(D, HP)
    wv = v_w.transpose(2, 1, 0, 3).reshape(D, E * HP).astype(_BF)
    wo = o_w.transpose(1, 0, 2, 3).reshape(E, HP, D).astype(_BF)
    pad = ((0, 0), (0, _GL - HE))
    wqs = jnp.concatenate([wq, jnp.pad(sel_o_w, pad)], axis=1).astype(_BF)
    wks = jnp.concatenate([wk, jnp.pad(sel_v_w, pad)], axis=1).astype(_BF)

    BS = 256
    nS = S // BS
    f32 = jnp.float32
    q_p, k_p, kpe_p, v_p, go_p = pl.pallas_call(
        functools.partial(_proj_body, H, E, P, D),
        grid=(nS,),
        in_specs=[
            pl.BlockSpec((BS, D), lambda i: (i, 0)),
            pl.BlockSpec((BS, D), lambda i: (i, 0)),
            pl.BlockSpec((BS, D), lambda i: (i, 0)),
            pl.BlockSpec((D, HP + _GL), lambda i: (0, 0)),
            pl.BlockSpec((D, HP + _GL), lambda i: (0, 0)),
            pl.BlockSpec((D, E * HP), lambda i: (0, 0)),
        ],
        out_specs=[
            pl.BlockSpec((H, BS, P), lambda i: (0, i, 0)),
            pl.BlockSpec((H, BS, P), lambda i: (0, i, 0)),
            pl.BlockSpec((H, BS, P), lambda i: (0, i, 0)),
            pl.BlockSpec((H, BS, P), lambda i: (0, i, 0)),
            pl.BlockSpec((BS, _GL), lambda i: (i, 0)),
        ],
        out_shape=[
            jax.ShapeDtypeStruct((H, S, P), f32),
            jax.ShapeDtypeStruct((H, S, P), f32),
            jax.ShapeDtypeStruct((H, S, P), f32),
            jax.ShapeDtypeStruct((H, S, P), f32),
            jax.ShapeDtypeStruct((S, _GL), f32),
        ],
    )(qs, ks, pes, wqs, wks, wv)

    BQ = 512
    nQ = S // BQ
    scale = 1.0 / float(P) ** 0.5
    ctx = pl.pallas_call(
        functools.partial(_attn_body, S, BQ, P, scale),
        grid=(H, nQ),
        in_specs=[
            pl.BlockSpec((1, BQ, P), lambda h, i: (h, i, 0)),
            pl.BlockSpec((1, S, P), lambda h, i: (h, 0, 0)),
            pl.BlockSpec((1, S, P), lambda h, i: (h, 0, 0)),
            pl.BlockSpec((1, S, P), lambda h, i: (h, 0, 0)),
        ],
        out_specs=pl.BlockSpec((1, BQ, P), lambda h, i: (h, i, 0)),
        out_shape=jax.ShapeDtypeStruct((H, S, P), f32),
    )(q_p, k_p, v_p, kpe_p)

    ctx2 = ctx.transpose(1, 0, 2).reshape(S, HP)

    BO = 512
    nO = S // BO
    out = pl.pallas_call(
        functools.partial(_out_body, H, E, P, D),
        grid=(nO,),
        in_specs=[
            pl.BlockSpec((BO, HP), lambda i: (i, 0)),
            pl.BlockSpec((BO, _GL), lambda i: (i, 0)),
            pl.BlockSpec((E, HP, D), lambda i: (0, 0, 0)),
        ],
        out_specs=pl.BlockSpec((BO, D), lambda i: (i, 0)),
        out_shape=jax.ShapeDtypeStruct((S, D), f32),
    )(ctx2, go_p, wo)

    return out.reshape(B, S, D)


# speedup vs baseline: 9.8119x; 1.1158x over previous
import functools

import jax
import jax.numpy as jnp
from jax.experimental import pallas as pl
from jax.experimental.pallas import tpu as pltpu

_BF = jnp.bfloat16
_GL = 128


def _dot(a, b):
    return jax.lax.dot_general(
        a.astype(_BF), b.astype(_BF), (((1,), (0,)), ((), ())),
        preferred_element_type=jnp.float32)


def _dot_t(a, b):
    return jax.lax.dot_general(
        a.astype(_BF), b.astype(_BF), (((1,), (1,)), ((), ())),
        preferred_element_type=jnp.float32)


def _spread(e, H, E, P, D):
    jr = jax.lax.broadcasted_iota(jnp.int32, (_GL, H * P), 0)
    jc = jax.lax.broadcasted_iota(jnp.int32, (_GL, H * P), 1)
    return jnp.where(jr == (jc // P) * E + e, 1.0, 0.0).astype(jnp.float32)


def _grp_sel(m, i, ms, is_, ok):
    better = ok & ((ms > m) | ((ms == m) & (is_ < i)))
    return jnp.where(better, ms, m), jnp.where(better, is_, i)


def _grp_argmax(val, p8, it8):
    W = val.shape[1]
    pm, pi = val, it8
    sm, si = val, it8
    for k in (1, 2, 4):
        pm_s = pltpu.roll(pm, k, 1)
        pi_s = pltpu.roll(pi, k, 1)
        pm, pi = _grp_sel(pm, pi, pm_s, pi_s, p8 >= k)
        sm_s = pltpu.roll(sm, W - k, 1)
        si_s = pltpu.roll(si, W - k, 1)
        sm, si = _grp_sel(sm, si, sm_s, si_s, p8 <= 7 - k)
    return _grp_sel(pm, pi, sm, si, pm == pm)


def _top2_gates(raw):
    val = 1.0 / (1.0 + jnp.exp(-raw))
    bs = raw.shape[0]
    lane = jax.lax.broadcasted_iota(jnp.int32, (bs, _GL), 1)
    p8 = jax.lax.rem(lane, 8)
    it8 = p8.astype(jnp.float32)
    p8f = it8
    m1, i1 = _grp_argmax(val, p8f, it8)
    msk1 = it8 == i1
    val2 = jnp.where(msk1, -1.0, val)
    m2, i2 = _grp_argmax(val2, p8f, it8)
    msk2 = it8 == i2
    return jnp.where(msk1, m1, 0.0) + jnp.where(msk2, m2, 0.0)


def _proj_body(H, E, P, D, qs_ref, ks_ref, pe_ref, wqs_ref, wks_ref, wv_ref,
               qo_ref, ko_ref, kpo_ref, vo_ref, go_ref):
    HP = H * P
    x_q = qs_ref[...]
    x_k = ks_ref[...]
    qs_big = _dot(x_q, wqs_ref[...])
    ks_big = _dot(x_k, wks_ref[...])
    kp_big = _dot(pe_ref[...], wks_ref[..., :HP])
    q_big = qs_big[:, :HP]
    k_big = ks_big[:, :HP]
    gv = _top2_gates(ks_big[:, HP:])
    go_ref[...] = _top2_gates(qs_big[:, HP:])
    v_all = _dot(x_k, wv_ref[...])
    v_acc = None
    for e in range(E):
        ge = _dot(gv, _spread(e, H, E, P, D))
        piece = v_all[:, e * HP:(e + 1) * HP] * ge
        v_acc = piece if v_acc is None else v_acc + piece
    for h in range(H):
        qo_ref[h] = q_big[:, h * P:(h + 1) * P].astype(_BF)
        ko_ref[h] = k_big[:, h * P:(h + 1) * P].astype(_BF)
        kpo_ref[h] = kp_big[:, h * P:(h + 1) * P].astype(_BF)
        vo_ref[h] = v_acc[:, h * P:(h + 1) * P].astype(_BF)


def _attn_body(S, BQ, P, scale, q_ref, k_ref, v_ref, kp_ref, o_ref):
    qb = pl.program_id(1)
    i0 = qb * BQ
    W = S + BQ
    q = q_ref[0]
    content = _dot_t(q, k_ref[0])
    kpw = kp_ref[0, pl.ds(S - i0 - BQ, W), :]
    pmat = _dot_t(q, kpw)
    pos = pltpu.roll(pmat, S + 1, 1, stride=1, stride_axis=0)[:, :S]
    it = jax.lax.broadcasted_iota(jnp.int32, (BQ, S), 1)
    ir = jax.lax.broadcasted_iota(jnp.int32, (BQ, S), 0) + i0
    x = jnp.where(it <= ir, (content + pos) * scale, -1e30)
    m = jnp.max(x, axis=1, keepdims=True)
    p = jnp.exp(x - m)
    att = p / jnp.sum(p, axis=1, keepdims=True)
    o_ref[0] = _dot(att, v_ref[0])


def _out_body(H, E, P, D, ctx_ref, go_ref, wo_ref, out_ref):
    ctx = ctx_ref[...]
    go = go_ref[...]
    acc = None
    for e in range(E):
        ge = _dot(go, _spread(e, H, E, P, D))
        term = _dot(ctx * ge, wo_ref[e])
        acc = term if acc is None else acc + term
    out_ref[...] = acc


def kernel(q_src, k_src, pe, q_w, k_w, v_w, o_w, sel_v_w, sel_o_w):
    B, S, D = q_src.shape
    H, _, P = q_w.shape
    E = v_w.shape[1]
    HP = H * P
    HE = H * E

    qs = q_src[0].astype(_BF)
    ks = k_src[0].astype(_BF)
    pes = pe[:S].astype(_BF)
    wq = q_w.transpose(1, 0, 2).reshape(D, HP)
    wk = k_w.transpose(1, 0, 2).reshape(D, HP)
    wv = v_w.transpose(2, 1, 0, 3).reshape(D, E * HP).astype(_BF)
    wo = o_w.transpose(1, 0, 2, 3).reshape(E, HP, D).astype(_BF)
    pad = ((0, 0), (0, _GL - HE))
    wqs = jnp.concatenate([wq, jnp.pad(sel_o_w, pad)], axis=1).astype(_BF)
    wks = jnp.concatenate([wk, jnp.pad(sel_v_w, pad)], axis=1).astype(_BF)

    BS = 256
    nS = S // BS
    f32 = jnp.float32
    q_p, k_p, kpe_p, v_p, go_p = pl.pallas_call(
        functools.partial(_proj_body, H, E, P, D),
        grid=(nS,),
        in_specs=[
            pl.BlockSpec((BS, D), lambda i: (i, 0)),
            pl.BlockSpec((BS, D), lambda i: (i, 0)),
            pl.BlockSpec((BS, D), lambda i: (i, 0)),
            pl.BlockSpec((D, HP + _GL), lambda i: (0, 0)),
            pl.BlockSpec((D, HP + _GL), lambda i: (0, 0)),
            pl.BlockSpec((D, E * HP), lambda i: (0, 0)),
        ],
        out_specs=[
            pl.BlockSpec((H, BS, P), lambda i: (0, i, 0)),
            pl.BlockSpec((H, BS, P), lambda i: (0, i, 0)),
            pl.BlockSpec((H, BS, P), lambda i: (0, i, 0)),
            pl.BlockSpec((H, BS, P), lambda i: (0, i, 0)),
            pl.BlockSpec((BS, _GL), lambda i: (i, 0)),
        ],
        out_shape=[
            jax.ShapeDtypeStruct((H, S, P), _BF),
            jax.ShapeDtypeStruct((H, S, P), _BF),
            jax.ShapeDtypeStruct((H, S, P), _BF),
            jax.ShapeDtypeStruct((H, S, P), _BF),
            jax.ShapeDtypeStruct((S, _GL), f32),
        ],
    )(qs, ks, pes, wqs, wks, wv)

    kpe_pad = jnp.pad(kpe_p, ((0, 0), (0, S), (0, 0)))

    BQ = 512
    nQ = S // BQ
    scale = 1.0 / float(P) ** 0.5
    ctx = pl.pallas_call(
        functools.partial(_attn_body, S, BQ, P, scale),
        grid=(H, nQ),
        in_specs=[
            pl.BlockSpec((1, BQ, P), lambda h, i: (h, i, 0)),
            pl.BlockSpec((1, S, P), lambda h, i: (h, 0, 0)),
            pl.BlockSpec((1, S, P), lambda h, i: (h, 0, 0)),
            pl.BlockSpec((1, 2 * S, P), lambda h, i: (h, 0, 0)),
        ],
        out_specs=pl.BlockSpec((1, BQ, P), lambda h, i: (h, i, 0)),
        out_shape=jax.ShapeDtypeStruct((H, S, P), f32),
    )(q_p, k_p, v_p, kpe_pad)

    ctx2 = ctx.transpose(1, 0, 2).reshape(S, HP)

    BO = 512
    nO = S // BO
    out = pl.pallas_call(
        functools.partial(_out_body, H, E, P, D),
        grid=(nO,),
        in_specs=[
            pl.BlockSpec((BO, HP), lambda i: (i, 0)),
            pl.BlockSpec((BO, _GL), lambda i: (i, 0)),
            pl.BlockSpec((E, HP, D), lambda i: (0, 0, 0)),
        ],
        out_specs=pl.BlockSpec((BO, D), lambda i: (i, 0)),
        out_shape=jax.ShapeDtypeStruct((S, D), f32),
    )(ctx2, go_p, wo)

    return out.reshape(B, S, D)


# speedup vs baseline: 10.6112x; 1.0815x over previous
import functools

import jax
import jax.numpy as jnp
from jax.experimental import pallas as pl
from jax.experimental.pallas import tpu as pltpu

_BF = jnp.bfloat16
_GL = 128


def _dot(a, b):
    return jax.lax.dot_general(
        a.astype(_BF), b.astype(_BF), (((1,), (0,)), ((), ())),
        preferred_element_type=jnp.float32)


def _dot_t(a, b):
    return jax.lax.dot_general(
        a.astype(_BF), b.astype(_BF), (((1,), (1,)), ((), ())),
        preferred_element_type=jnp.float32)


def _spread(e, H, E, P):
    jr = jax.lax.broadcasted_iota(jnp.int32, (_GL, H * P), 0)
    jc = jax.lax.broadcasted_iota(jnp.int32, (_GL, H * P), 1)
    return jnp.where(jr == (jc // P) * E + e, 1.0, 0.0).astype(jnp.float32)


def _grp_sel(m, i, ms, is_, ok):
    better = ok & ((ms > m) | ((ms == m) & (is_ < i)))
    return jnp.where(better, ms, m), jnp.where(better, is_, i)


def _grp_argmax(val, p8, it8):
    W = val.shape[1]
    pm, pi = val, it8
    sm, si = val, it8
    for k in (1, 2, 4):
        pm_s = pltpu.roll(pm, k, 1)
        pi_s = pltpu.roll(pi, k, 1)
        pm, pi = _grp_sel(pm, pi, pm_s, pi_s, p8 >= k)
        sm_s = pltpu.roll(sm, W - k, 1)
        si_s = pltpu.roll(si, W - k, 1)
        sm, si = _grp_sel(sm, si, sm_s, si_s, p8 <= 7 - k)
    return _grp_sel(pm, pi, sm, si, pm == pm)


def _top2_gates(raw):
    val = 1.0 / (1.0 + jnp.exp(-raw))
    bs = raw.shape[0]
    lane = jax.lax.broadcasted_iota(jnp.int32, (bs, _GL), 1)
    it8 = jax.lax.rem(lane, 8).astype(jnp.float32)
    m1, i1 = _grp_argmax(val, it8, it8)
    msk1 = it8 == i1
    val2 = jnp.where(msk1, -1.0, val)
    m2, i2 = _grp_argmax(val2, it8, it8)
    msk2 = it8 == i2
    return jnp.where(msk1, m1, 0.0) + jnp.where(msk2, m2, 0.0)


def _proj_body(H, E, P, qs_ref, ks_ref, pe_ref, wqs_ref, wks_ref, wv_ref,
               qo_ref, ko_ref, kpo_ref, vo_ref, go_ref):
    HP = H * P
    x_q = qs_ref[...]
    x_k = ks_ref[...]
    qs_big = _dot(x_q, wqs_ref[...])
    ks_big = _dot(x_k, wks_ref[...])
    kp_big = _dot(pe_ref[...], wks_ref[..., :HP])
    q_big = qs_big[:, :HP]
    k_big = ks_big[:, :HP]
    gv = _top2_gates(ks_big[:, HP:])
    go_ref[...] = _top2_gates(qs_big[:, HP:])
    v_all = _dot(x_k, wv_ref[...])
    v_acc = None
    for e in range(E):
        ge = _dot(gv, _spread(e, H, E, P))
        piece = v_all[:, e * HP:(e + 1) * HP] * ge
        v_acc = piece if v_acc is None else v_acc + piece
    for h in range(H):
        qo_ref[h] = q_big[:, h * P:(h + 1) * P].astype(_BF)
        ko_ref[h] = k_big[:, h * P:(h + 1) * P].astype(_BF)
        kpo_ref[h] = kp_big[:, h * P:(h + 1) * P].astype(_BF)
        vo_ref[h] = v_acc[:, h * P:(h + 1) * P].astype(_BF)


def _attn_body(S, BQ, H, E, P, D, scale,
               q_ref, k_ref, v_ref, kp_ref, go_ref, wo_ref, out_ref, ctx_ref):
    HP = H * P
    qb = pl.program_id(0)
    h = pl.program_id(1)
    i0 = qb * BQ
    W = S + BQ
    q = q_ref[0]
    content = _dot_t(q, k_ref[0])
    kpw = kp_ref[0, pl.ds(S - i0 - BQ, W), :]
    pmat = _dot_t(q, kpw).astype(_BF)
    pos = pltpu.roll(pmat, S + 1, 1, stride=1, stride_axis=0)[:, :S]
    it = jax.lax.broadcasted_iota(jnp.int32, (BQ, S), 1)
    ir = jax.lax.broadcasted_iota(jnp.int32, (BQ, S), 0) + i0
    x = jnp.where(it <= ir, (content + pos.astype(jnp.float32)) * scale, -1e30)
    m = jnp.max(x, axis=1, keepdims=True)
    p = jnp.exp(x - m)
    att = p / jnp.sum(p, axis=1, keepdims=True)
    ctx_ref[h] = _dot(att, v_ref[0]).astype(_BF)

    @pl.when(h == H - 1)
    def _out():
        ctx = jnp.concatenate([ctx_ref[hh] for hh in range(H)], axis=1)
        go = go_ref[...]
        acc = None
        for e in range(E):
            ge = _dot(go, _spread(e, H, E, P))
            term = _dot(ctx * ge, wo_ref[e])
            acc = term if acc is None else acc + term
        out_ref[...] = acc


def kernel(q_src, k_src, pe, q_w, k_w, v_w, o_w, sel_v_w, sel_o_w):
    B, S, D = q_src.shape
    H, _, P = q_w.shape
    E = v_w.shape[1]
    HP = H * P
    HE = H * E

    qs = q_src[0].astype(_BF)
    ks = k_src[0].astype(_BF)
    pes = pe[:S].astype(_BF)
    wq = q_w.astype(_BF).transpose(1, 0, 2).reshape(D, HP)
    wk = k_w.astype(_BF).transpose(1, 0, 2).reshape(D, HP)
    wv = v_w.astype(_BF).transpose(2, 1, 0, 3).reshape(D, E * HP)
    wo = o_w.astype(_BF).transpose(1, 0, 2, 3).reshape(E, HP, D)
    pad = ((0, 0), (0, _GL - HE))
    wqs = jnp.concatenate([wq, jnp.pad(sel_o_w.astype(_BF), pad)], axis=1)
    wks = jnp.concatenate([wk, jnp.pad(sel_v_w.astype(_BF), pad)], axis=1)

    BS = 256
    nS = S // BS
    f32 = jnp.float32
    q_p, k_p, kpe_p, v_p, go_p = pl.pallas_call(
        functools.partial(_proj_body, H, E, P),
        grid=(nS,),
        in_specs=[
            pl.BlockSpec((BS, D), lambda i: (i, 0)),
            pl.BlockSpec((BS, D), lambda i: (i, 0)),
            pl.BlockSpec((BS, D), lambda i: (i, 0)),
            pl.BlockSpec((D, HP + _GL), lambda i: (0, 0)),
            pl.BlockSpec((D, HP + _GL), lambda i: (0, 0)),
            pl.BlockSpec((D, E * HP), lambda i: (0, 0)),
        ],
        out_specs=[
            pl.BlockSpec((H, BS, P), lambda i: (0, i, 0)),
            pl.BlockSpec((H, BS, P), lambda i: (0, i, 0)),
            pl.BlockSpec((H, BS, P), lambda i: (0, i, 0)),
            pl.BlockSpec((H, BS, P), lambda i: (0, i, 0)),
            pl.BlockSpec((BS, _GL), lambda i: (i, 0)),
        ],
        out_shape=[
            jax.ShapeDtypeStruct((H, S, P), _BF),
            jax.ShapeDtypeStruct((H, S, P), _BF),
            jax.ShapeDtypeStruct((H, 2 * S, P), _BF),
            jax.ShapeDtypeStruct((H, S, P), _BF),
            jax.ShapeDtypeStruct((S, _GL), f32),
        ],
    )(qs, ks, pes, wqs, wks, wv)

    BQ = 512
    nQ = S // BQ
    scale = 1.0 / float(P) ** 0.5
    out = pl.pallas_call(
        functools.partial(_attn_body, S, BQ, H, E, P, D, scale),
        grid=(nQ, H),
        in_specs=[
            pl.BlockSpec((1, BQ, P), lambda i, h: (h, i, 0)),
            pl.BlockSpec((1, S, P), lambda i, h: (h, 0, 0)),
            pl.BlockSpec((1, S, P), lambda i, h: (h, 0, 0)),
            pl.BlockSpec((1, 2 * S, P), lambda i, h: (h, 0, 0)),
            pl.BlockSpec((BQ, _GL), lambda i, h: (i, 0)),
            pl.BlockSpec((E, HP, D), lambda i, h: (0, 0, 0)),
        ],
        out_specs=pl.BlockSpec((BQ, D), lambda i, h: (i, 0)),
        out_shape=jax.ShapeDtypeStruct((S, D), f32),
        scratch_shapes=[pltpu.VMEM((H, BQ, P), _BF)],
    )(q_p, k_p, v_p, kpe_p, go_p, wo)

    return out.reshape(B, S, D)


# speedup vs baseline: 12.2297x; 1.1525x over previous
import functools

import jax
import jax.numpy as jnp
from jax.experimental import pallas as pl
from jax.experimental.pallas import tpu as pltpu

_BF = jnp.bfloat16
_GL = 128


def _dot(a, b):
    return jax.lax.dot_general(
        a.astype(_BF), b.astype(_BF), (((1,), (0,)), ((), ())),
        preferred_element_type=jnp.float32)


def _dot_t(a, b):
    return jax.lax.dot_general(
        a.astype(_BF), b.astype(_BF), (((1,), (1,)), ((), ())),
        preferred_element_type=jnp.float32)


def _spread(e, H, E, P):
    jr = jax.lax.broadcasted_iota(jnp.int32, (_GL, H * P), 0)
    jc = jax.lax.broadcasted_iota(jnp.int32, (_GL, H * P), 1)
    return jnp.where(jr == (jc // P) * E + e, 1.0, 0.0).astype(jnp.float32)


def _grp_sel(m, i, ms, is_, ok):
    better = ok & ((ms > m) | ((ms == m) & (is_ < i)))
    return jnp.where(better, ms, m), jnp.where(better, is_, i)


def _grp_argmax(val, p8, it8):
    W = val.shape[1]
    pm, pi = val, it8
    sm, si = val, it8
    for k in (1, 2, 4):
        pm_s = pltpu.roll(pm, k, 1)
        pi_s = pltpu.roll(pi, k, 1)
        pm, pi = _grp_sel(pm, pi, pm_s, pi_s, p8 >= k)
        sm_s = pltpu.roll(sm, W - k, 1)
        si_s = pltpu.roll(si, W - k, 1)
        sm, si = _grp_sel(sm, si, sm_s, si_s, p8 <= 7 - k)
    return _grp_sel(pm, pi, sm, si, pm == pm)


def _top2_gates(raw):
    val = 1.0 / (1.0 + jnp.exp(-raw))
    bs = raw.shape[0]
    lane = jax.lax.broadcasted_iota(jnp.int32, (bs, _GL), 1)
    it8 = jax.lax.rem(lane, 8).astype(jnp.float32)
    m1, i1 = _grp_argmax(val, it8, it8)
    msk1 = it8 == i1
    val2 = jnp.where(msk1, -1.0, val)
    m2, i2 = _grp_argmax(val2, it8, it8)
    msk2 = it8 == i2
    return jnp.where(msk1, m1, 0.0) + jnp.where(msk2, m2, 0.0)


def _proj_body(H, E, P, qs_ref, ks_ref, pe_ref, wqs_ref, wks_ref, wv_ref,
               qo_ref, ko_ref, kpo_ref, vo_ref, go_ref):
    HP = H * P
    x_q = qs_ref[...]
    x_k = ks_ref[...]
    qs_big = _dot(x_q, wqs_ref[...])
    ks_big = _dot(x_k, wks_ref[...])
    kp_big = _dot(pe_ref[...], wks_ref[..., :HP])
    q_big = qs_big[:, :HP]
    k_big = ks_big[:, :HP]
    gv = _top2_gates(ks_big[:, HP:])
    go_ref[...] = _top2_gates(qs_big[:, HP:])
    v_all = _dot(x_k, wv_ref[...])
    v_acc = None
    for e in range(E):
        ge = _dot(gv, _spread(e, H, E, P))
        piece = v_all[:, e * HP:(e + 1) * HP] * ge
        v_acc = piece if v_acc is None else v_acc + piece
    for h in range(H):
        qo_ref[h] = q_big[:, h * P:(h + 1) * P].astype(_BF)
        ko_ref[h] = k_big[:, h * P:(h + 1) * P].astype(_BF)
        kpo_ref[h] = kp_big[:, h * P:(h + 1) * P].astype(_BF)
        vo_ref[h] = v_acc[:, h * P:(h + 1) * P].astype(_BF)


def _attn_body(S, BQ, KL, i0, H, E, P, D, scale,
               q_ref, k_ref, v_ref, kp_ref, go_ref, wo_ref, out_ref, ctx_ref):
    HP = H * P
    h = pl.program_id(0)
    W = KL + BQ
    q = q_ref[0]
    content = _dot_t(q, k_ref[0])
    kpw = kp_ref[0, pl.ds(S - KL, W), :]
    pmat = _dot_t(q, kpw).astype(_BF)
    pos = pltpu.roll(pmat, W - BQ + 1, 1, stride=1, stride_axis=0)[:, :KL]
    it = jax.lax.broadcasted_iota(jnp.int32, (BQ, KL), 1)
    ir = jax.lax.broadcasted_iota(jnp.int32, (BQ, KL), 0) + i0
    x = jnp.where(it <= ir, (content + pos.astype(jnp.float32)) * scale, -1e30)
    m = jnp.max(x, axis=1, keepdims=True)
    p = jnp.exp(x - m)
    att = p / jnp.sum(p, axis=1, keepdims=True)
    ctx_ref[h] = _dot(att, v_ref[0]).astype(_BF)

    @pl.when(h == H - 1)
    def _out():
        ctx = jnp.concatenate([ctx_ref[hh] for hh in range(H)], axis=1)
        go = go_ref[...]
        acc = None
        for e in range(E):
            ge = _dot(go, _spread(e, H, E, P))
            term = _dot(ctx * ge, wo_ref[e])
            acc = term if acc is None else acc + term
        out_ref[...] = acc


def kernel(q_src, k_src, pe, q_w, k_w, v_w, o_w, sel_v_w, sel_o_w):
    B, S, D = q_src.shape
    H, _, P = q_w.shape
    E = v_w.shape[1]
    HP = H * P
    HE = H * E

    qs = q_src[0].astype(_BF)
    ks = k_src[0].astype(_BF)
    pes = pe[:S].astype(_BF)
    wq = q_w.astype(_BF).transpose(1, 0, 2).reshape(D, HP)
    wk = k_w.astype(_BF).transpose(1, 0, 2).reshape(D, HP)
    wv = v_w.astype(_BF).transpose(2, 1, 0, 3).reshape(D, E * HP)
    wo = o_w.astype(_BF).transpose(1, 0, 2, 3).reshape(E, HP, D)
    pad = ((0, 0), (0, _GL - HE))
    wqs = jnp.concatenate([wq, jnp.pad(sel_o_w.astype(_BF), pad)], axis=1)
    wks = jnp.concatenate([wk, jnp.pad(sel_v_w.astype(_BF), pad)], axis=1)

    BS = 256
    nS = S // BS
    f32 = jnp.float32
    q_p, k_p, kpe_p, v_p, go_p = pl.pallas_call(
        functools.partial(_proj_body, H, E, P),
        grid=(nS,),
        in_specs=[
            pl.BlockSpec((BS, D), lambda i: (i, 0)),
            pl.BlockSpec((BS, D), lambda i: (i, 0)),
            pl.BlockSpec((BS, D), lambda i: (i, 0)),
            pl.BlockSpec((D, HP + _GL), lambda i: (0, 0)),
            pl.BlockSpec((D, HP + _GL), lambda i: (0, 0)),
            pl.BlockSpec((D, E * HP), lambda i: (0, 0)),
        ],
        out_specs=[
            pl.BlockSpec((H, BS, P), lambda i: (0, i, 0)),
            pl.BlockSpec((H, BS, P), lambda i: (0, i, 0)),
            pl.BlockSpec((H, BS, P), lambda i: (0, i, 0)),
            pl.BlockSpec((H, BS, P), lambda i: (0, i, 0)),
            pl.BlockSpec((BS, _GL), lambda i: (i, 0)),
        ],
        out_shape=[
            jax.ShapeDtypeStruct((H, S, P), _BF),
            jax.ShapeDtypeStruct((H, S, P), _BF),
            jax.ShapeDtypeStruct((H, 2 * S, P), _BF),
            jax.ShapeDtypeStruct((H, S, P), _BF),
            jax.ShapeDtypeStruct((S, _GL), f32),
        ],
    )(qs, ks, pes, wqs, wks, wv)

    BQ = 512
    nQ = S // BQ
    scale = 1.0 / float(P) ** 0.5
    outs = []
    for j in range(nQ):
        KL = (j + 1) * BQ
        out_j = pl.pallas_call(
            functools.partial(_attn_body, S, BQ, KL, j * BQ, H, E, P, D,
                              scale),
            grid=(H,),
            in_specs=[
                pl.BlockSpec((1, BQ, P), lambda h, j=j: (h, j, 0)),
                pl.BlockSpec((1, KL, P), lambda h: (h, 0, 0)),
                pl.BlockSpec((1, KL, P), lambda h: (h, 0, 0)),
                pl.BlockSpec((1, 2 * S, P), lambda h: (h, 0, 0)),
                pl.BlockSpec((BQ, _GL), lambda h, j=j: (j, 0)),
                pl.BlockSpec((E, HP, D), lambda h: (0, 0, 0)),
            ],
            out_specs=pl.BlockSpec((BQ, D), lambda h: (0, 0)),
            out_shape=jax.ShapeDtypeStruct((BQ, D), f32),
            scratch_shapes=[pltpu.VMEM((H, BQ, P), _BF)],
        )(q_p, k_p, v_p, kpe_p, go_p, wo)
        outs.append(out_j)
    out = jnp.concatenate(outs, axis=0)

    return out.reshape(B, S, D)
